# baseline (speedup 1.0000x reference)
import functools
import math

import jax
import jax.numpy as jnp
from jax.experimental import pallas as pl
from jax.experimental.pallas import tpu as pltpu

N = 2048
H = 128
C = 8
GROUPS = 4
EPS = 1e-5
RADIUS = 9.0

BI = 256
BJ = 512
NI = N // BI
NJ = N // BJ
NG = 2 + C


def _leaky(x):
    return jnp.where(x >= 0, x, 0.2 * x)


def _group_norm(x, gamma, beta):
    gs = H // GROUPS
    cols = []
    for g in range(GROUPS):
        sub = x[:, g * gs:(g + 1) * gs]
        m = jnp.mean(sub)
        v = jnp.mean((sub - m) * (sub - m))
        cols.append((sub - m) * jax.lax.rsqrt(v + EPS))
    y = jnp.concatenate(cols, axis=1)
    return y * gamma + beta


def _mlp_in(x, w1, b1, w2, b2, gw, gb):
    f = jnp.dot(x, w1, preferred_element_type=jnp.float32)
    f = _leaky(f + b1)
    f = jnp.dot(f, w2, preferred_element_type=jnp.float32)
    f = _leaky(f + b2)
    return _group_norm(f, gw, gb)


def _mlp_out(u, x, wo1, bo1, wo2, bo2, gw, gb, l1, l1b, l2, l2b, lt, ltb):
    o = jnp.dot(u, wo1, preferred_element_type=jnp.float32)
    o = _leaky(o + bo1)
    o = jnp.dot(o, wo2, preferred_element_type=jnp.float32)
    o = _leaky(o + bo2)
    o = _group_norm(o, gw, gb)
    xi = jnp.dot(o, l1, preferred_element_type=jnp.float32)
    xi = jnp.maximum(xi + l1b, 0.0)
    xi = jnp.dot(xi, l2, preferred_element_type=jnp.float32) + l2b
    xn = jnp.dot(x, lt, preferred_element_type=jnp.float32)
    return xn + ltb + xi


def _build_ubig(gp, w1k, a1e, cb1r, u_ref):
    p3 = gp[:, 0:3]
    pn2 = gp[:, 3:4]
    nr = gp[:, 4:7]
    nuv9 = gp[:, 7:16]
    q9 = gp[:, 16:25]
    a2d = jnp.dot(nuv9, w1k, preferred_element_type=jnp.float32)
    bias = cb1r - jnp.dot(q9, a1e, preferred_element_type=jnp.float32)
    z1 = jnp.zeros((N, 1), jnp.float32)
    z3 = jnp.zeros((N, 3), jnp.float32)
    o1 = jnp.ones((N, 1), jnp.float32)
    u_ref[0] = jnp.concatenate([-2.0 * p3, o1, z3, pn2], axis=1)
    u_ref[1] = jnp.concatenate([z3, z1, -nr, 2.0 * o1], axis=1)
    for c in range(C):
        u_ref[2 + c] = jnp.concatenate(
            [a2d[:, 3 * c:3 * c + 3], z3, z1, bias[:, c:c + 1]], axis=1)


def _head_body(x_ref, w1_ref, b1_ref, w2_ref, b2_ref, gw_ref, gb_ref,
               gp_ref, w1k_ref, a1e_ref, cb1_ref, f_ref, u_ref):
    f_ref[...] = _mlp_in(x_ref[...], w1_ref[...], b1_ref[...], w2_ref[...],
                         b2_ref[...], gw_ref[...], gb_ref[...])
    _build_ubig(gp_ref[...], w1k_ref[...], a1e_ref[...], cb1_ref[...], u_ref)


def _mid_body(u_ref, x_ref, wo1_ref, bo1_ref, wo2_ref, bo2_ref, gow_ref,
              gob_ref, l1_ref, l1b_ref, l2_ref, l2b_ref, lt_ref, ltb_ref,
              w1_ref, b1_ref, w2_ref, b2_ref, gw_ref, gb_ref,
              gp_ref, w1k_ref, a1e_ref, cb1_ref,
              xn_ref, f_ref, ub_ref):
    xn = _mlp_out(u_ref[...], x_ref[...], wo1_ref[...], bo1_ref[...],
                  wo2_ref[...], bo2_ref[...], gow_ref[...], gob_ref[...],
                  l1_ref[...], l1b_ref[...], l2_ref[...], l2b_ref[...],
                  lt_ref[...], ltb_ref[...])
    xn_ref[...] = xn
    f_ref[...] = _mlp_in(xn, w1_ref[...], b1_ref[...], w2_ref[...],
                         b2_ref[...], gw_ref[...], gb_ref[...])
    _build_ubig(gp_ref[...], w1k_ref[...], a1e_ref[...], cb1_ref[...], ub_ref)


def _tail_body(u_ref, x_ref, wo1_ref, bo1_ref, wo2_ref, bo2_ref, gow_ref,
               gob_ref, l1_ref, l1b_ref, l2_ref, l2b_ref, lt_ref, ltb_ref,
               o_ref):
    o_ref[...] = _mlp_out(u_ref[...], x_ref[...], wo1_ref[...], bo1_ref[...],
                          wo2_ref[...], bo2_ref[...], gow_ref[...],
                          gob_ref[...], l1_ref[...], l1b_ref[...],
                          l2_ref[...], l2b_ref[...], lt_ref[...], ltb_ref[...])


def _pair_body(u_ref, jp_ref, f_ref, a2t_ref, cb2_ref, o_ref):
    j = pl.program_id(1)
    jp = jp_ref[...]
    fj = f_ref[...]

    um = u_ref[...].reshape(NG * BI, 8)
    tt = jnp.dot(um, jp, preferred_element_type=jnp.float32)

    sq = tt[0:BI]
    e = tt[BI:2 * BI]
    w = jnp.exp(-sq * (e * e))

    acc = jnp.dot(w, fj, preferred_element_type=jnp.float32) * cb2_ref[...]
    for c in range(C):
        r = jnp.maximum(tt[(2 + c) * BI:(3 + c) * BI], 0.0) * w
        acc += (jnp.dot(r, fj, preferred_element_type=jnp.float32)
                * a2t_ref[c:c + 1, :])

    @pl.when(j == 0)
    def _():
        o_ref[...] = acc

    @pl.when(j != 0)
    def _():
        o_ref[...] += acc


_f32 = jnp.float32

_head_call = pl.pallas_call(
    _head_body,
    out_shape=[jax.ShapeDtypeStruct((N, H), _f32),
               jax.ShapeDtypeStruct((NG, N, 8), _f32)],
)

_mid_call = pl.pallas_call(
    _mid_body,
    out_shape=[jax.ShapeDtypeStruct((N, H), _f32),
               jax.ShapeDtypeStruct((N, H), _f32),
               jax.ShapeDtypeStruct((NG, N, 8), _f32)],
)

_tail_call = pl.pallas_call(
    _tail_body,
    out_shape=jax.ShapeDtypeStruct((N, H), _f32),
)

_pair_call = pl.pallas_call(
    _pair_body,
    grid=(NI, NJ),
    in_specs=[
        pl.BlockSpec((NG, BI, 8), lambda i, j: (0, i, 0)),
        pl.BlockSpec((8, BJ), lambda i, j: (0, j)),
        pl.BlockSpec((BJ, H), lambda i, j: (j, 0)),
        pl.BlockSpec((C, H), lambda i, j: (0, 0)),
        pl.BlockSpec((1, H), lambda i, j: (0, 0)),
    ],
    out_specs=pl.BlockSpec((BI, H), lambda i, j: (i, 0)),
    out_shape=jax.ShapeDtypeStruct((N, H), _f32),
    compiler_params=pltpu.CompilerParams(
        dimension_semantics=("parallel", "arbitrary"),
    ),
)


def kernel(features, points, normals, ranges, win1, bin1, win2, bin2, gni_w,
           gni_b, a1, cb1, a2, cb2, wo1, bo1, wo2, bo2, gno_w, gno_b, ll1_w,
           ll1_b, ll2_w, ll2_b, lt_w, lt_b):
    p = points * (1.0 / (math.sqrt(2.0) * RADIUS))
    pn2 = jnp.sum(p * p, axis=1, keepdims=True)
    nrm = normals[:, 0, :]
    ocol = jnp.ones((N, 1), jnp.float32)

    jpack = jnp.concatenate([p, pn2, nrm, ocol], axis=1).T

    nuv9 = normals.reshape(N, 9)
    q9 = nuv9 * jnp.tile(p, (1, 3))
    gpack = jnp.concatenate([p, pn2, nrm, nuv9, q9], axis=1)

    eye3 = jnp.eye(3, dtype=jnp.float32)
    w1k0 = jnp.kron(a1[0], eye3).T
    w1k1 = jnp.kron(a1[1], eye3).T
    a1e0 = jnp.repeat(a1[0], 3, axis=1).T
    a1e1 = jnp.repeat(a1[1], 3, axis=1).T

    f0, ubig0 = _head_call(
        features, win1[0].T, bin1[0][None, :], win2[0].T, bin2[0][None, :],
        gni_w[0][None, :], gni_b[0][None, :], gpack, w1k0, a1e0,
        cb1[0][None, :])

    u0 = _pair_call(ubig0, jpack, f0, a2[0].T, cb2[0][None, :])

    x1, f1, ubig1 = _mid_call(
        u0, features, wo1[0].T, bo1[0][None, :], wo2[0].T, bo2[0][None, :],
        gno_w[0][None, :], gno_b[0][None, :], ll1_w[0].T, ll1_b[0][None, :],
        ll2_w[0].T, ll2_b[0][None, :], lt_w[0].T, lt_b[0][None, :],
        win1[1].T, bin1[1][None, :], win2[1].T, bin2[1][None, :],
        gni_w[1][None, :], gni_b[1][None, :], gpack, w1k1, a1e1,
        cb1[1][None, :])

    u1 = _pair_call(ubig1, jpack, f1, a2[1].T, cb2[1][None, :])

    x2 = _tail_call(
        u1, x1, wo1[1].T, bo1[1][None, :], wo2[1].T, bo2[1][None, :],
        gno_w[1][None, :], gno_b[1][None, :], ll1_w[1].T, ll1_b[1][None, :],
        ll2_w[1].T, ll2_b[1][None, :], lt_w[1].T, lt_b[1][None, :])
    return x2
